# baseline (speedup 1.0000x reference)
import functools

import jax
import jax.numpy as jnp
from jax import lax
from jax.experimental import pallas as pl
from jax.experimental.pallas import tpu as pltpu
from jax.experimental.pallas import tpu_sc as plsc

VOCAB = 1000000
DIM = 64
N = 4096 * 200
NC, NS = 2, 16
NW = NC * NS
BPW = N // NW
C = 128
NCHUNK = BPW // C


def _make_merged_gather():
    mesh = plsc.VectorSubcoreMesh(core_axis_name="c", subcore_axis_name="s")

    @functools.partial(
        pl.kernel,
        out_type=jax.ShapeDtypeStruct((N, DIM), jnp.float32),
        mesh=mesh,
        scratch_types=[
            pltpu.VMEM((BPW,), jnp.int32),
            pltpu.VMEM((C, DIM), jnp.float32),
            pltpu.VMEM((C, DIM), jnp.float32),
            pltpu.VMEM((2, 16), jnp.float32),
            pltpu.SemaphoreType.DMA,
        ],
        compiler_params=pltpu.CompilerParams(use_tc_tiling_on_sc=False),
    )
    def merged_gather(idx_hbm, base_hbm, mod_hbm, coef_hbm, out_hbm,
                      idx_v, brows, mrows, cvec, sem):
        wid = lax.axis_index("s") * NC + lax.axis_index("c")
        row0 = wid * BPW
        pltpu.sync_copy(coef_hbm, cvec)
        pltpu.sync_copy(idx_hbm.at[pl.ds(row0, BPW)], idx_v)
        c0 = cvec[0, :]
        c1 = cvec[1, :]

        def chunk_body(g, carry):
            off = g * C
            gb = pltpu.async_copy(base_hbm.at[idx_v.at[pl.ds(off, C)]], brows, sem)
            gm = pltpu.async_copy(mod_hbm.at[idx_v.at[pl.ds(off, C)]], mrows, sem)
            gb.wait()
            gm.wait()

            def row_body(r, rc):
                for j in range(DIM // 16):
                    s = pl.ds(j * 16, 16)
                    brows[r, s] = c0 * brows[r, s] + c1 * mrows[r, s]
                return rc

            lax.fori_loop(0, C, row_body, 0, unroll=2)
            pltpu.sync_copy(brows, out_hbm.at[pl.ds(row0 + off, C)])
            return carry

        lax.fori_loop(0, NCHUNK, chunk_body, 0)

    return merged_gather


_merged_gather = _make_merged_gather()


def kernel(input, base_weight, mod_weight_0, mod_weight_1, merging_coefficients):
    del mod_weight_1
    idx = input.reshape(-1).astype(jnp.int32)
    coefs = jnp.broadcast_to(
        merging_coefficients.astype(jnp.float32)[:, None], (2, 16)
    )
    out = _merged_gather(idx, base_weight, mod_weight_0, coefs)
    return out.reshape(input.shape + (DIM,))


# speedup vs baseline: 1.1282x; 1.1282x over previous
import functools

import jax
import jax.numpy as jnp
from jax import lax
from jax.experimental import pallas as pl
from jax.experimental.pallas import tpu as pltpu
from jax.experimental.pallas import tpu_sc as plsc

VOCAB = 1000000
DIM = 64
N = 4096 * 200
NC, NS = 2, 16
NW = NC * NS
BPW = N // NW
C = 128
NCHUNK = BPW // C
NB = 4


def _make_merged_gather():
    mesh = plsc.VectorSubcoreMesh(core_axis_name="c", subcore_axis_name="s")

    @functools.partial(
        pl.kernel,
        out_type=jax.ShapeDtypeStruct((N, DIM), jnp.float32),
        mesh=mesh,
        scratch_types=[
            pltpu.VMEM((BPW,), jnp.int32),
        ]
        + [pltpu.VMEM((C, DIM), jnp.float32)] * (3 * NB)
        + [
            pltpu.VMEM((2, 16), jnp.float32),
        ]
        + [pltpu.SemaphoreType.DMA] * (2 * NB),
        compiler_params=pltpu.CompilerParams(use_tc_tiling_on_sc=False),
    )
    def merged_gather(idx_hbm, base_hbm, mod_hbm, coef_hbm, out_hbm,
                      idx_v, *scratch):
        brows = scratch[:NB]
        mrows = scratch[NB:2 * NB]
        orows = scratch[2 * NB:3 * NB]
        cvec = scratch[3 * NB]
        gsem = scratch[3 * NB + 1:3 * NB + 1 + NB]
        osem = scratch[3 * NB + 1 + NB:]
        wid = lax.axis_index("s") * NC + lax.axis_index("c")
        row0 = wid * BPW
        pltpu.sync_copy(coef_hbm, cvec)
        pltpu.sync_copy(idx_hbm.at[pl.ds(row0, BPW)], idx_v)
        c0 = cvec[0, :]
        c1 = cvec[1, :]

        def start_gather(g, b):
            idx_slice = idx_v.at[pl.ds(g * C, C)]
            pltpu.async_copy(base_hbm.at[idx_slice], brows[b], gsem[b])
            pltpu.async_copy(mod_hbm.at[idx_slice], mrows[b], gsem[b])

        def wait_gather(g, b):
            idx_slice = idx_v.at[pl.ds(g * C, C)]
            pltpu.make_async_copy(base_hbm.at[idx_slice], brows[b], gsem[b]).wait()
            pltpu.make_async_copy(mod_hbm.at[idx_slice], mrows[b], gsem[b]).wait()

        def out_slice(g):
            return out_hbm.at[pl.ds(row0 + g * C, C)]

        for b in range(NB):
            start_gather(b, b)

        def outer(w, carry):
            for b in range(NB):
                g = w * NB + b
                wait_gather(g, b)

                @pl.when(w > 0)
                def _():
                    pltpu.make_async_copy(orows[b], out_slice(g - NB), osem[b]).wait()

                def row_body(r, rc):
                    for j in range(DIM // 16):
                        s = pl.ds(j * 16, 16)
                        orows[b][r, s] = c0 * brows[b][r, s] + c1 * mrows[b][r, s]
                    return rc

                lax.fori_loop(0, C, row_body, 0, unroll=4)

                @pl.when(g + NB < NCHUNK)
                def _():
                    start_gather(g + NB, b)

                pltpu.async_copy(orows[b], out_slice(g), osem[b])
            return carry

        lax.fori_loop(0, NCHUNK // NB, outer, 0)

        for b in range(NB):
            g = NCHUNK - NB + b
            pltpu.make_async_copy(orows[b], out_slice(g), osem[b]).wait()

    return merged_gather


_merged_gather = _make_merged_gather()


def kernel(input, base_weight, mod_weight_0, mod_weight_1, merging_coefficients):
    del mod_weight_1
    idx = input.reshape(-1).astype(jnp.int32)
    coefs = jnp.broadcast_to(
        merging_coefficients.astype(jnp.float32)[:, None], (2, 16)
    )
    out = _merged_gather(idx, base_weight, mod_weight_0, coefs)
    return out.reshape(input.shape + (DIM,))


# speedup vs baseline: 1.3652x; 1.2100x over previous
import functools

import jax
import jax.numpy as jnp
from jax import lax
from jax.experimental import pallas as pl
from jax.experimental.pallas import tpu as pltpu
from jax.experimental.pallas import tpu_sc as plsc

VOCAB = 1000000
DIM = 64
N = 4096 * 200
NC, NS = 2, 16
NW = NC * NS
BPW = N // NW
C = 128
NCHUNK = BPW // C
NB = 4


def _make_merged_gather():
    mesh = plsc.VectorSubcoreMesh(core_axis_name="c", subcore_axis_name="s")

    @functools.partial(
        pl.kernel,
        out_type=jax.ShapeDtypeStruct((N // 2, 2 * DIM), jnp.float32),
        mesh=mesh,
        scratch_types=[
            pltpu.VMEM((BPW,), jnp.int32),
        ]
        + [pltpu.VMEM((C, 2 * DIM), jnp.float32)] * NB
        + [
            pltpu.VMEM((2, 16), jnp.float32),
        ]
        + [pltpu.SemaphoreType.DMA] * (2 * NB),
    )
    def merged_gather(idx_hbm, pair_hbm, coef_hbm, out_hbm, idx_v, *scratch):
        lrows = scratch[:NB]
        cvec = scratch[NB]
        gsem = scratch[NB + 1:NB + 1 + NB]
        osem = scratch[NB + 1 + NB:]
        wid = lax.axis_index("s") * NC + lax.axis_index("c")
        row0 = wid * BPW
        pltpu.sync_copy(coef_hbm, cvec)
        pltpu.sync_copy(idx_hbm.at[pl.ds(row0, BPW)], idx_v)
        c0 = cvec[0, :]
        c1 = cvec[1, :]

        def start_gather(g, b):
            idx_slice = idx_v.at[pl.ds(g * C, C)]
            pltpu.async_copy(pair_hbm.at[idx_slice], lrows[b], gsem[b])

        def wait_gather(g, b):
            idx_slice = idx_v.at[pl.ds(g * C, C)]
            pltpu.make_async_copy(pair_hbm.at[idx_slice], lrows[b], gsem[b]).wait()

        def out_slice(g):
            off = pl.multiple_of((row0 + g * C) // 2, C // 2)
            return out_hbm.at[pl.ds(off, C // 2)]

        for b in range(NB):
            start_gather(b, b)

        def outer(w, carry):
            for b in range(NB):
                g = w * NB + b
                wait_gather(g, b)

                def row_body(r, rc):
                    for j in range(DIM // 16):
                        s = pl.ds(j * 16, 16)
                        sm = pl.ds(DIM + j * 16, 16)
                        m0 = c0 * lrows[b][2 * r, s] + c1 * lrows[b][2 * r, sm]
                        m1 = c0 * lrows[b][2 * r + 1, s] + c1 * lrows[b][2 * r + 1, sm]
                        lrows[b][r, s] = m0
                        lrows[b][r, sm] = m1
                    return rc

                lax.fori_loop(0, C // 2, row_body, 0, unroll=4)

                merged = lrows[b].at[pl.ds(0, C // 2)]
                pltpu.async_copy(merged, out_slice(g), osem[b])

                @pl.when(g + NB < NCHUNK)
                def _():
                    pltpu.make_async_copy(merged, out_slice(g), osem[b]).wait()
                    start_gather(g + NB, b)
            return carry

        lax.fori_loop(0, NCHUNK // NB, outer, 0)

        for b in range(NB):
            g = NCHUNK - NB + b
            merged = lrows[b].at[pl.ds(0, C // 2)]
            pltpu.make_async_copy(merged, out_slice(g), osem[b]).wait()

    return merged_gather


_merged_gather = _make_merged_gather()


def kernel(input, base_weight, mod_weight_0, mod_weight_1, merging_coefficients):
    del mod_weight_1
    idx = input.reshape(-1).astype(jnp.int32)
    pair = jnp.concatenate([base_weight, mod_weight_0], axis=1)
    coefs = jnp.broadcast_to(
        merging_coefficients.astype(jnp.float32)[:, None], (2, 16)
    )
    out = _merged_gather(idx, pair, coefs)
    return out.reshape(input.shape + (DIM,))
